# baseline (speedup 1.0000x reference)
import jax
import jax.numpy as jnp
from jax.experimental import pallas as pl
from jax.experimental.pallas import tpu as pltpu

_ROWS = 16384
_COLS = 2048
_BLOCK_ROWS = 512
_NB = _ROWS // _BLOCK_ROWS
_K = 7


def _quant_body(nl_ref, x_ref, o_ref, res_ref, accmin_ref, accmax_ref,
                mm_ref):
    p = pl.program_id(0)
    i = pl.program_id(1)

    @pl.when(p == 0)
    def _reduce_phase():
        @pl.when(i == 0)
        def _init():
            accmin_ref[...] = jnp.full((8, _COLS), 3.4e38, jnp.float32)
            accmax_ref[...] = jnp.full((8, _COLS), -3.4e38, jnp.float32)

        x = x_ref[...]
        mn = accmin_ref[...]
        mx = accmax_ref[...]
        for u in range(_BLOCK_ROWS // 8):
            s = x[u * 8:(u + 1) * 8, :]
            mn = jnp.minimum(mn, s)
            mx = jnp.maximum(mx, s)
        accmin_ref[...] = mn
        accmax_ref[...] = mx

        @pl.when(i < _K)
        def _stash():
            res_ref[pl.ds(i * _BLOCK_ROWS, _BLOCK_ROWS), :] = x

    @pl.when(p == 1)
    def _quantize_phase():
        @pl.when(i == 0)
        def _finalize():
            mm_ref[0] = jnp.min(accmin_ref[...])
            mm_ref[1] = jnp.max(accmax_ref[...])

        nl = nl_ref[0]
        rng = mm_ref[1] - mm_ref[0]
        scale = rng / nl
        inv_scale = nl / rng

        half = (i % 2) * _BLOCK_ROWS

        @pl.when(i < _K)
        def _from_vmem():
            r = res_ref[pl.ds(i * _BLOCK_ROWS, _BLOCK_ROWS), :]
            o_ref[pl.ds(half, _BLOCK_ROWS), :] = jnp.round(r * inv_scale) * scale

        @pl.when(i >= _K)
        def _from_hbm():
            o_ref[pl.ds(half, _BLOCK_ROWS), :] = (
                jnp.round(x_ref[...] * inv_scale) * scale)


def kernel(input, bits):
    nlevels = (jnp.exp2(bits.astype(jnp.float32)) - 1.0
               if hasattr(bits, "astype")
               else jnp.float32(2.0 ** bits - 1.0))
    nlevels = jnp.reshape(nlevels, (1,))
    x2 = input.reshape(_ROWS, _COLS)

    def x_map(p, i):
        return (jnp.where(p == 0, i, jnp.where(i < _K, _NB - 1, i)), 0)

    out = pl.pallas_call(
        _quant_body,
        grid=(2, _NB),
        in_specs=[
            pl.BlockSpec(memory_space=pltpu.SMEM),
            pl.BlockSpec((_BLOCK_ROWS, _COLS), x_map),
        ],
        out_specs=pl.BlockSpec((2 * _BLOCK_ROWS, _COLS),
                               lambda p, i: (p * (i // 2), 0)),
        out_shape=jax.ShapeDtypeStruct((_ROWS, _COLS), jnp.float32),
        scratch_shapes=[pltpu.VMEM((_K * _BLOCK_ROWS, _COLS), jnp.float32),
                        pltpu.VMEM((8, _COLS), jnp.float32),
                        pltpu.VMEM((8, _COLS), jnp.float32),
                        pltpu.SMEM((2,), jnp.float32)],
    )(nlevels, x2)
    return out.reshape(input.shape)


# speedup vs baseline: 1.1241x; 1.1241x over previous
import jax
import jax.numpy as jnp
from jax import lax
from jax.experimental import pallas as pl
from jax.experimental.pallas import tpu as pltpu

_ROWS = 16384
_COLS = 2048
_BR = 512
_NB = _ROWS // _BR
_K = 10
_ST = _BR // 8


def _body(nl_ref, x_hbm, o_hbm, res, in0, in1, o0, o1, sin, sout):
    def in_dst(i):
        if i < _K:
            return res.at[pl.ds(i * _BR, _BR), :]
        return in0 if i % 2 == 0 else in1

    def in_copy(i):
        return pltpu.make_async_copy(
            x_hbm.at[pl.ds(i * _BR, _BR), :], in_dst(i), sin.at[i % 2])

    def out_copy(i, ob):
        return pltpu.make_async_copy(
            ob, o_hbm.at[pl.ds(i * _BR, _BR), :], sout.at[i % 2])

    in_copy(0).start()
    in_copy(1).start()

    big = jnp.float32(3.4e38)
    mn = jnp.full((8, _COLS), big, jnp.float32)
    mx = jnp.full((8, _COLS), -big, jnp.float32)

    for i in range(_NB):
        in_copy(i).wait()
        if i + 2 < _NB:
            in_copy(i + 2).start()
        src = in_dst(i)

        def stripe(j, c, src=src):
            m, M = c
            s = src[pl.ds(j * 8, 8), :]
            return jnp.minimum(m, s), jnp.maximum(M, s)

        mn, mx = lax.fori_loop(0, _ST, stripe, (mn, mx))

    gmin = jnp.min(mn)
    gmax = jnp.max(mx)
    nl = nl_ref[0]
    rng = gmax - gmin
    scale = rng / nl
    inv_scale = nl / rng

    in_copy(_K).start()
    in_copy(_K + 1).start()

    for i in range(_NB):
        if i >= _K:
            in_copy(i).wait()
        if i >= 2:
            out_copy(i - 2, o0 if i % 2 == 0 else o1).wait()
        src = in_dst(i)
        ob = o0 if i % 2 == 0 else o1

        @pl.loop(0, _ST)
        def _(j, src=src, ob=ob):
            ob[pl.ds(j * 8, 8), :] = (
                jnp.round(src[pl.ds(j * 8, 8), :] * inv_scale) * scale)

        out_copy(i, ob).start()
        if i >= _K and i + 2 < _NB:
            in_copy(i + 2).start()

    out_copy(_NB - 2, o0).wait()
    out_copy(_NB - 1, o1).wait()


def kernel(input, bits):
    nlevels = (jnp.exp2(bits.astype(jnp.float32)) - 1.0
               if hasattr(bits, "astype")
               else jnp.float32(2.0 ** bits - 1.0))
    nlevels = jnp.reshape(nlevels, (1,))
    x2 = input.reshape(_ROWS, _COLS)

    out = pl.pallas_call(
        _body,
        in_specs=[
            pl.BlockSpec(memory_space=pltpu.SMEM),
            pl.BlockSpec(memory_space=pl.ANY),
        ],
        out_specs=pl.BlockSpec(memory_space=pl.ANY),
        out_shape=jax.ShapeDtypeStruct((_ROWS, _COLS), jnp.float32),
        scratch_shapes=[pltpu.VMEM((_K * _BR, _COLS), jnp.float32),
                        pltpu.VMEM((_BR, _COLS), jnp.float32),
                        pltpu.VMEM((_BR, _COLS), jnp.float32),
                        pltpu.VMEM((_BR, _COLS), jnp.float32),
                        pltpu.VMEM((_BR, _COLS), jnp.float32),
                        pltpu.SemaphoreType.DMA((2,)),
                        pltpu.SemaphoreType.DMA((2,))],
    )(nlevels, x2)
    return out.reshape(input.shape)


# speedup vs baseline: 1.1513x; 1.0242x over previous
import jax
import jax.numpy as jnp
from jax import lax
from jax.experimental import pallas as pl
from jax.experimental.pallas import tpu as pltpu

_ROWS = 16384
_COLS = 2048
_BR = 512
_NB = _ROWS // _BR
_K = 10
_ST = _BR // 8


def _body(nl_ref, x_hbm, o_hbm, res, in0, in1, o0, o1, sin, sout):
    def in_dst(i):
        if i < _K:
            return res.at[pl.ds(i * _BR, _BR), :]
        if i == _K:
            return o0
        if i == _K + 1:
            return o1
        return in0 if i % 2 == 0 else in1

    def in_copy(i):
        return pltpu.make_async_copy(
            x_hbm.at[pl.ds(i * _BR, _BR), :], in_dst(i), sin.at[i % 2])

    def out_copy(i, ob, slot):
        return pltpu.make_async_copy(
            ob, o_hbm.at[pl.ds(i * _BR, _BR), :], sout.at[slot])

    in_copy(0).start()
    in_copy(1).start()

    big = jnp.float32(3.4e38)
    mn = jnp.full((8, _COLS), big, jnp.float32)
    mx = jnp.full((8, _COLS), -big, jnp.float32)

    for i in range(_NB):
        in_copy(i).wait()
        if i + 2 < _NB:
            in_copy(i + 2).start()
        src = in_dst(i)

        def stripe(j, c, src=src):
            m, M = c
            s = src[pl.ds(j * 8, 8), :]
            return jnp.minimum(m, s), jnp.maximum(M, s)

        mn, mx = lax.fori_loop(0, _ST, stripe, (mn, mx))

    gmin = jnp.min(mn)
    gmax = jnp.max(mx)
    nl = nl_ref[0]
    rng = gmax - gmin
    scale = rng / nl
    inv_scale = nl / rng

    order = [_K, _K + 1] + list(range(_K)) + list(range(_K + 2, _NB))

    ring = [b for b in order if b >= _K + 2]
    if len(ring) >= 1:
        in_copy(ring[0]).start()
    if len(ring) >= 2:
        in_copy(ring[1]).start()

    last_out = [None, None]
    started = 2
    for pos, b in enumerate(order):
        slot = pos % 2
        ob = o0 if slot == 0 else o1
        if b >= _K + 2:
            in_copy(b).wait()
        if last_out[slot] is not None:
            out_copy(last_out[slot], ob, slot).wait()
        src = ob if b in (_K, _K + 1) else in_dst(b)

        @pl.loop(0, _ST)
        def _(j, src=src, ob=ob):
            ob[pl.ds(j * 8, 8), :] = (
                jnp.round(src[pl.ds(j * 8, 8), :] * inv_scale) * scale)

        out_copy(b, ob, slot).start()
        last_out[slot] = b
        if b >= _K + 2 and started < len(ring):
            in_copy(ring[started]).start()
            started += 1

    out_copy(last_out[0], o0, 0).wait()
    out_copy(last_out[1], o1, 1).wait()


def kernel(input, bits):
    nlevels = (jnp.exp2(bits.astype(jnp.float32)) - 1.0
               if hasattr(bits, "astype")
               else jnp.float32(2.0 ** bits - 1.0))
    nlevels = jnp.reshape(nlevels, (1,))
    x2 = input.reshape(_ROWS, _COLS)

    out = pl.pallas_call(
        _body,
        in_specs=[
            pl.BlockSpec(memory_space=pltpu.SMEM),
            pl.BlockSpec(memory_space=pl.ANY),
        ],
        out_specs=pl.BlockSpec(memory_space=pl.ANY),
        out_shape=jax.ShapeDtypeStruct((_ROWS, _COLS), jnp.float32),
        scratch_shapes=[pltpu.VMEM((_K * _BR, _COLS), jnp.float32),
                        pltpu.VMEM((_BR, _COLS), jnp.float32),
                        pltpu.VMEM((_BR, _COLS), jnp.float32),
                        pltpu.VMEM((_BR, _COLS), jnp.float32),
                        pltpu.VMEM((_BR, _COLS), jnp.float32),
                        pltpu.SemaphoreType.DMA((2,)),
                        pltpu.SemaphoreType.DMA((2,))],
    )(nlevels, x2)
    return out.reshape(input.shape)


# speedup vs baseline: 1.1668x; 1.0135x over previous
import jax
import jax.numpy as jnp
from jax import lax
from jax.experimental import pallas as pl
from jax.experimental.pallas import tpu as pltpu

_ROWS = 16384
_COLS = 2048
_BR = 512
_NB = _ROWS // _BR
_K = 10
_ST = _BR // 8


def _body(nl_ref, x_hbm, o_hbm, res, in0, in1, o0, o1, sin, sout):
    def in_dst(i):
        if i < _K:
            return res.at[pl.ds(i * _BR, _BR), :]
        if i == _K:
            return o0
        if i == _K + 1:
            return o1
        return in0 if i % 2 == 0 else in1

    def in_copy(i):
        return pltpu.make_async_copy(
            x_hbm.at[pl.ds(i * _BR, _BR), :], in_dst(i), sin.at[i % 2])

    def out_copy(i, ob, slot):
        return pltpu.make_async_copy(
            ob, o_hbm.at[pl.ds(i * _BR, _BR), :], sout.at[slot])

    in_copy(0).start()
    in_copy(1).start()

    big = jnp.float32(3.4e38)
    mn = jnp.full((8, _COLS), big, jnp.float32)
    mx = jnp.full((8, _COLS), -big, jnp.float32)

    for i in range(_NB):
        in_copy(i).wait()
        if i + 2 < _NB:
            in_copy(i + 2).start()
        src = in_dst(i)

        def stripe(j, c, src=src):
            m, M = c
            s = src[pl.ds(j * 8, 8), :]
            return jnp.minimum(m, s), jnp.maximum(M, s)

        mn, mx = lax.fori_loop(0, _ST, stripe, (mn, mx))

    gmin = jnp.min(mn)
    gmax = jnp.max(mx)
    nl = nl_ref[0]
    rng = gmax - gmin
    scale = rng / nl
    inv_scale = nl / rng

    order = ([_K, _K + 1, _K + 2, _K + 3] + list(range(_K))
             + list(range(_K + 4, _NB)))
    ring = list(range(_K + 4, _NB))

    last_out = [None, None]
    started = 0
    for pos, b in enumerate(order):
        slot = pos % 2
        ob = o0 if slot == 0 else o1
        if b >= _K + 4:
            in_copy(b).wait()
        if last_out[slot] is not None:
            out_copy(last_out[slot], ob, slot).wait()
        src = ob if b in (_K, _K + 1) else in_dst(b)

        @pl.loop(0, _ST)
        def _(j, src=src, ob=ob):
            ob[pl.ds(j * 8, 8), :] = (
                jnp.round(src[pl.ds(j * 8, 8), :] * inv_scale) * scale)

        out_copy(b, ob, slot).start()
        last_out[slot] = b
        if b in (_K + 2, _K + 3) or (b >= _K + 4 and started < len(ring)):
            if started < len(ring):
                in_copy(ring[started]).start()
                started += 1

    out_copy(last_out[0], o0, 0).wait()
    out_copy(last_out[1], o1, 1).wait()


def kernel(input, bits):
    nlevels = (jnp.exp2(bits.astype(jnp.float32)) - 1.0
               if hasattr(bits, "astype")
               else jnp.float32(2.0 ** bits - 1.0))
    nlevels = jnp.reshape(nlevels, (1,))
    x2 = input.reshape(_ROWS, _COLS)

    out = pl.pallas_call(
        _body,
        in_specs=[
            pl.BlockSpec(memory_space=pltpu.SMEM),
            pl.BlockSpec(memory_space=pl.ANY),
        ],
        out_specs=pl.BlockSpec(memory_space=pl.ANY),
        out_shape=jax.ShapeDtypeStruct((_ROWS, _COLS), jnp.float32),
        scratch_shapes=[pltpu.VMEM((_K * _BR, _COLS), jnp.float32),
                        pltpu.VMEM((_BR, _COLS), jnp.float32),
                        pltpu.VMEM((_BR, _COLS), jnp.float32),
                        pltpu.VMEM((_BR, _COLS), jnp.float32),
                        pltpu.VMEM((_BR, _COLS), jnp.float32),
                        pltpu.SemaphoreType.DMA((2,)),
                        pltpu.SemaphoreType.DMA((2,))],
    )(nlevels, x2)
    return out.reshape(input.shape)
